# baseline (speedup 1.0000x reference)
import functools

import jax
import jax.numpy as jnp
from jax import lax
from jax.experimental import pallas as pl
from jax.experimental.pallas import tpu as pltpu
from jax.experimental.pallas import tpu_sc as plsc

B, S, D, E, K = 4, 8192, 768, 8, 2
N = B * S
T = 4096
NC, NS, L = 2, 16, 16
NW = NC * NS
TOK_W = N // NW


def _router_block(x_ref, wt_ref, weights_t_ref):
    x = x_ref[...]
    wt = wt_ref[...]
    logits_t = lax.dot_general(
        wt, x, (((1,), (1,)), ((), ())), preferred_element_type=jnp.float32
    )
    m1 = jnp.max(logits_t, axis=0, keepdims=True)
    ex = jnp.exp(logits_t - m1)
    weights_t_ref[...] = ex / jnp.sum(ex, axis=0, keepdims=True)


@functools.partial(
    pl.kernel,
    mesh=plsc.VectorSubcoreMesh(core_axis_name="c", subcore_axis_name="s"),
    out_type=[
        jax.ShapeDtypeStruct((K, N), jnp.float32),
        jax.ShapeDtypeStruct((K, N), jnp.int32),
    ],
    scratch_types=[
        pltpu.VMEM((E, TOK_W), jnp.float32),
        pltpu.VMEM((K, TOK_W), jnp.float32),
        pltpu.VMEM((K, TOK_W), jnp.int32),
    ],
)
def _sc_top2(w_hbm, topw_hbm, tope_hbm, w_v, topw_v, tope_v):
    wid = lax.axis_index("s") * NC + lax.axis_index("c")
    base = wid * TOK_W
    pltpu.sync_copy(w_hbm.at[:, pl.ds(base, TOK_W)], w_v)

    def body(g, carry):
        col = g * L
        rows = [w_v[e, pl.ds(col, L)] for e in range(E)]
        m1 = rows[0]
        id1 = jnp.zeros((L,), jnp.int32)
        for e in range(1, E):
            gt = rows[e] > m1
            m1 = jnp.where(gt, rows[e], m1)
            id1 = jnp.where(gt, jnp.int32(e), id1)
        m2 = jnp.full((L,), -1.0, jnp.float32)
        id2 = jnp.zeros((L,), jnp.int32)
        for e in range(E):
            take = (rows[e] > m2) & (id1 != jnp.int32(e))
            m2 = jnp.where(take, rows[e], m2)
            id2 = jnp.where(take, jnp.int32(e), id2)
        ssum = m1 + m2
        topw_v[0, pl.ds(col, L)] = m1 / ssum
        topw_v[1, pl.ds(col, L)] = m2 / ssum
        tope_v[0, pl.ds(col, L)] = id1
        tope_v[1, pl.ds(col, L)] = id2
        return carry

    lax.fori_loop(0, TOK_W // L, body, 0)
    pltpu.sync_copy(topw_v, topw_hbm.at[:, pl.ds(base, TOK_W)])
    pltpu.sync_copy(tope_v, tope_hbm.at[:, pl.ds(base, TOK_W)])


@jax.jit
def kernel(x, W):
    xf = x.reshape(N, D)
    wt = W.T
    weights_t = pl.pallas_call(
        _router_block,
        grid=(N // T,),
        in_specs=[
            pl.BlockSpec((T, D), lambda i: (i, 0)),
            pl.BlockSpec((E, D), lambda i: (0, 0)),
        ],
        out_specs=pl.BlockSpec((E, T), lambda i: (0, i)),
        out_shape=jax.ShapeDtypeStruct((E, N), jnp.float32),
    )(xf, wt)
    topw_t, tope_t = _sc_top2(weights_t)
    return (
        weights_t.T.reshape(B, S, E),
        topw_t.T.reshape(B, S, K),
        tope_t.T.reshape(B, S, K),
    )
